# Initial kernel scaffold; baseline (speedup 1.0000x reference)
import jax
import jax.numpy as jnp
from jax.experimental import pallas as pl


def kernel(a_p, block_ids, block_counts, x, w1, b1, w2, b2):
    raise NotImplementedError("write your pallas kernel here")


# speedup vs baseline: 1.3850x; 1.3850x over previous
import functools

import jax
import jax.numpy as jnp
from jax.experimental import pallas as pl
from jax.experimental.pallas import tpu as pltpu

LANE = 128


def _round_up(x, m):
    return ((x + m - 1) // m) * m


def _pass1_kernel(ids_ref, cnts_ref, a_ref, x_ref, w1_ref, b1_ref, w2_ref,
                  o_ref, acc_ref, *, tile_k):
    i = pl.program_id(0)
    k = pl.program_id(1)

    @pl.when(k == 0)
    def _():
        acc_ref[...] = jnp.zeros_like(acc_ref)

    @pl.when(k < cnts_ref[i])
    def _():
        b = ids_ref[i, k]
        acc_ref[...] += jnp.dot(
            a_ref[...], x_ref[pl.ds(b * tile_k, tile_k), :],
            preferred_element_type=jnp.float32)

    @pl.when(k == pl.num_programs(1) - 1)
    def _():
        h = jnp.maximum(
            jnp.dot(acc_ref[...], w1_ref[...],
                    preferred_element_type=jnp.float32) + b1_ref[...], 0.0)
        o_ref[...] = jnp.dot(
            h.astype(w2_ref.dtype), w2_ref[...],
            preferred_element_type=jnp.float32).astype(o_ref.dtype)


def _pass2_kernel(ids_ref, cnts_ref, a_ref, hw_ref, b2_ref, o_ref, acc_ref, *,
                  tile_k, num_classes):
    i = pl.program_id(0)
    k = pl.program_id(1)

    @pl.when(k == 0)
    def _():
        acc_ref[...] = jnp.zeros_like(acc_ref)

    @pl.when(k < cnts_ref[i])
    def _():
        b = ids_ref[i, k]
        acc_ref[...] += jnp.dot(
            a_ref[...], hw_ref[pl.ds(b * tile_k, tile_k), :],
            preferred_element_type=jnp.float32)

    @pl.when(k == pl.num_programs(1) - 1)
    def _():
        z = acc_ref[...] + b2_ref[...]
        col = jax.lax.broadcasted_iota(jnp.int32, z.shape, 1)
        z = jnp.where(col < num_classes, z, -1e30)
        m = jnp.max(z, axis=1, keepdims=True)
        s = z - m
        lse = jnp.log(jnp.sum(jnp.exp(s), axis=1, keepdims=True))
        o_ref[...] = (s - lse).astype(o_ref.dtype)


def kernel(a_p, block_ids, block_counts, x, w1, b1, w2, b2):
    tile_m = 256
    tile_k = 256
    n, f_in = x.shape
    hidden = w1.shape[1]
    num_classes = w2.shape[1]

    n_pad = a_p.shape[0]
    f_pad = _round_up(f_in, LANE)
    h_pad = _round_up(hidden, LANE)
    c_pad = _round_up(num_classes, LANE)
    grid_m = n_pad // tile_m
    max_kb = block_ids.shape[1]

    x_p = jnp.zeros((n_pad, f_pad), jnp.bfloat16).at[:n, :f_in].set(
        x.astype(jnp.bfloat16))
    w1_p = jnp.zeros((f_pad, h_pad), jnp.float32).at[:f_in, :hidden].set(
        w1.astype(jnp.float32))
    w2_p = jnp.zeros((h_pad, c_pad), jnp.bfloat16).at[:hidden, :num_classes].set(
        w2.astype(jnp.bfloat16))
    b1_p = jnp.zeros((1, h_pad), jnp.float32).at[:, :hidden].set(
        b1.astype(jnp.float32)[None, :])
    b2_p = jnp.zeros((1, c_pad), jnp.float32).at[:, :num_classes].set(
        b2.astype(jnp.float32)[None, :])

    cparams = pltpu.CompilerParams(
        dimension_semantics=("parallel", "arbitrary"),
        vmem_limit_bytes=64 * 1024 * 1024,
    )

    def a_map(i, k, ids, cnts):
        kk = jnp.minimum(k, jnp.maximum(cnts[i] - 1, 0))
        return (i, ids[i, kk])

    n_sched = grid_m * max_kb

    hw2 = pl.pallas_call(
        functools.partial(_pass1_kernel, tile_k=tile_k),
        out_shape=jax.ShapeDtypeStruct((n_pad, c_pad), jnp.bfloat16),
        grid_spec=pltpu.PrefetchScalarGridSpec(
            num_scalar_prefetch=2,
            grid=(grid_m, max_kb),
            in_specs=[
                pl.BlockSpec((tile_m, tile_k), a_map),
                pl.BlockSpec((n_pad, f_pad), lambda i, k, ids, cnts: (0, 0)),
                pl.BlockSpec((f_pad, h_pad), lambda i, k, ids, cnts: (0, 0)),
                pl.BlockSpec((1, h_pad), lambda i, k, ids, cnts: (0, 0)),
                pl.BlockSpec((h_pad, c_pad), lambda i, k, ids, cnts: (0, 0)),
            ],
            out_specs=pl.BlockSpec((tile_m, c_pad),
                                   lambda i, k, ids, cnts: (i, 0)),
            scratch_shapes=[pltpu.VMEM((tile_m, f_pad), jnp.float32)],
        ),
        compiler_params=cparams,
        cost_estimate=pl.CostEstimate(
            flops=2 * n_sched * tile_m * tile_k * f_pad
            + 2 * n_pad * (f_pad * h_pad + h_pad * c_pad),
            transcendentals=0,
            bytes_accessed=2 * (n_sched * tile_m * tile_k + n_pad * f_pad
                                + n_pad * c_pad) + 4 * f_pad * h_pad,
        ),
    )(block_ids, block_counts, a_p, x_p, w1_p, b1_p, w2_p)

    out_p = pl.pallas_call(
        functools.partial(_pass2_kernel, tile_k=tile_k,
                          num_classes=num_classes),
        out_shape=jax.ShapeDtypeStruct((n_pad, c_pad), jnp.float32),
        grid_spec=pltpu.PrefetchScalarGridSpec(
            num_scalar_prefetch=2,
            grid=(grid_m, max_kb),
            in_specs=[
                pl.BlockSpec((tile_m, tile_k), a_map),
                pl.BlockSpec((n_pad, c_pad), lambda i, k, ids, cnts: (0, 0)),
                pl.BlockSpec((1, c_pad), lambda i, k, ids, cnts: (0, 0)),
            ],
            out_specs=pl.BlockSpec((tile_m, c_pad),
                                   lambda i, k, ids, cnts: (i, 0)),
            scratch_shapes=[pltpu.VMEM((tile_m, c_pad), jnp.float32)],
        ),
        compiler_params=cparams,
        cost_estimate=pl.CostEstimate(
            flops=2 * n_sched * tile_m * tile_k * c_pad + 5 * n_pad * c_pad,
            transcendentals=2 * n_pad * c_pad,
            bytes_accessed=2 * (n_sched * tile_m * tile_k + n_pad * c_pad)
            + 4 * n_pad * c_pad,
        ),
    )(block_ids, block_counts, a_p, hw2, b2_p)

    return out_p[:n, :num_classes]
